# baseline (speedup 1.0000x reference)
import jax
import jax.numpy as jnp
from jax.experimental import pallas as pl
from jax.experimental.pallas import tpu as pltpu

_ACTION_S_IDX = 68
_ACTION_E_IDX = 88
_FULL_ACTION_DIM = 28
_PAD = 128
_BM = 1024


def _fused_body(obs_ref, prev_ref, w1_ref, b1_ref, w2_ref, b2_ref, w3_ref,
                t_ref, tb_ref, out_ref):
    bf16 = jnp.bfloat16
    obs = obs_ref[...]
    col = jax.lax.broadcasted_iota(jnp.int32, obs.shape, 1)
    in_seg = (col >= _ACTION_S_IDX) & (col < _ACTION_E_IDX)
    x = jnp.where(in_seg, prev_ref[...], obs).astype(bf16)
    h = jnp.tanh(jnp.dot(x, w1_ref[...].astype(bf16),
                         preferred_element_type=jnp.float32) + b1_ref[...])
    h = jnp.tanh(jnp.dot(h.astype(bf16), w2_ref[...].astype(bf16),
                         preferred_element_type=jnp.float32) + b2_ref[...])
    w3t = jnp.dot(w3_ref[...], t_ref[...], preferred_element_type=jnp.float32)
    res = jnp.dot(h.astype(bf16), w3t.astype(bf16),
                  preferred_element_type=jnp.float32) + tb_ref[...]
    out_ref[...] = res[:, :out_ref.shape[1]]


def kernel(obs, prev_full_action_wk, W1, b1, W2, b2, W3, b3,
           walking_action_out_indices, walking_offsets_indices,
           walking_offsets, walking_defaults, keep_mask):
    B, D = obs.shape
    H = W1.shape[1]
    nact = W3.shape[1]
    nkeep = keep_mask.shape[0]
    outw = nkeep + 4
    f32 = jnp.float32

    prev_row = jax.lax.dynamic_update_slice(
        jnp.zeros((1, D), f32), prev_full_action_wk.astype(f32),
        (0, _ACTION_S_IDX))

    M = jnp.zeros((nact, _FULL_ACTION_DIM), f32)
    M = M.at[jnp.arange(nact), walking_action_out_indices].set(0.1)
    c = jnp.zeros((_FULL_ACTION_DIM,), f32)
    c = c.at[walking_offsets_indices].set(walking_defaults)
    c = c.at[walking_action_out_indices].set(walking_offsets)
    T = jnp.zeros((_PAD, _PAD), f32).at[:nact, :nkeep].set(M[:, keep_mask])
    W3p = jnp.zeros((H, _PAD), f32).at[:, :nact].set(W3)
    b3p = jnp.zeros((_PAD,), f32).at[:nact].set(b3)
    tb = (b3p @ T + jnp.zeros((_PAD,), f32).at[:nkeep].set(c[keep_mask]))
    tb = tb.reshape(1, _PAD)

    bm = min(_BM, B)
    out = pl.pallas_call(
        _fused_body,
        grid=(pl.cdiv(B, bm),),
        in_specs=[
            pl.BlockSpec((bm, D), lambda i: (i, 0)),
            pl.BlockSpec((1, D), lambda i: (0, 0)),
            pl.BlockSpec((D, H), lambda i: (0, 0)),
            pl.BlockSpec((1, H), lambda i: (0, 0)),
            pl.BlockSpec((H, H), lambda i: (0, 0)),
            pl.BlockSpec((1, H), lambda i: (0, 0)),
            pl.BlockSpec((H, _PAD), lambda i: (0, 0)),
            pl.BlockSpec((_PAD, _PAD), lambda i: (0, 0)),
            pl.BlockSpec((1, _PAD), lambda i: (0, 0)),
        ],
        out_specs=pl.BlockSpec((bm, outw), lambda i: (i, 0)),
        out_shape=jax.ShapeDtypeStruct((B, outw), f32),
        compiler_params=pltpu.CompilerParams(
            dimension_semantics=("arbitrary",)),
    )(obs, prev_row, W1, b1.reshape(1, H), W2, b2.reshape(1, H), W3p, T, tb)
    return out


# speedup vs baseline: 1.2052x; 1.2052x over previous
import jax
import jax.numpy as jnp
from jax.experimental import pallas as pl
from jax.experimental.pallas import tpu as pltpu

_ACTION_S_IDX = 68
_ACTION_E_IDX = 88
_FULL_ACTION_DIM = 28
_PAD = 128
_BM = 2048


def _fused_body(obs_ref, prev_ref, w1_ref, b1_ref, w2_ref, b2_ref, w3t_ref,
                tb_ref, out_ref):
    bf16 = jnp.bfloat16
    obs = obs_ref[...]
    col = jax.lax.broadcasted_iota(jnp.int32, obs.shape, 1)
    in_seg = (col >= _ACTION_S_IDX) & (col < _ACTION_E_IDX)
    x = jnp.where(in_seg, prev_ref[...], obs).astype(bf16)
    h = jnp.tanh(jnp.dot(x, w1_ref[...],
                         preferred_element_type=jnp.float32) + b1_ref[...])
    h = jnp.tanh(jnp.dot(h.astype(bf16), w2_ref[...],
                         preferred_element_type=jnp.float32) + b2_ref[...])
    res = jnp.dot(h.astype(bf16), w3t_ref[...],
                  preferred_element_type=jnp.float32) + tb_ref[...]
    out_ref[...] = res[:, :out_ref.shape[1]]


def kernel(obs, prev_full_action_wk, W1, b1, W2, b2, W3, b3,
           walking_action_out_indices, walking_offsets_indices,
           walking_offsets, walking_defaults, keep_mask):
    B, D = obs.shape
    H = W1.shape[1]
    nact = W3.shape[1]
    nkeep = keep_mask.shape[0]
    outw = nkeep + 4
    f32 = jnp.float32
    bf16 = jnp.bfloat16

    prev_row = jax.lax.dynamic_update_slice(
        jnp.zeros((1, D), f32), prev_full_action_wk.astype(f32),
        (0, _ACTION_S_IDX))

    M = jnp.zeros((nact, _FULL_ACTION_DIM), f32)
    M = M.at[jnp.arange(nact), walking_action_out_indices].set(0.1)
    c = jnp.zeros((_FULL_ACTION_DIM,), f32)
    c = c.at[walking_offsets_indices].set(walking_defaults)
    c = c.at[walking_action_out_indices].set(walking_offsets)
    T = jnp.zeros((nact, _PAD), f32).at[:, :nkeep].set(M[:, keep_mask])
    W3T = (W3 @ T).astype(bf16)
    tb = (b3 @ T + jnp.zeros((_PAD,), f32).at[:nkeep].set(c[keep_mask]))
    tb = tb.reshape(1, _PAD)

    bm = min(_BM, B)
    out = pl.pallas_call(
        _fused_body,
        grid=(pl.cdiv(B, bm),),
        in_specs=[
            pl.BlockSpec((bm, D), lambda i: (i, 0)),
            pl.BlockSpec((1, D), lambda i: (0, 0)),
            pl.BlockSpec((D, H), lambda i: (0, 0)),
            pl.BlockSpec((1, H), lambda i: (0, 0)),
            pl.BlockSpec((H, H), lambda i: (0, 0)),
            pl.BlockSpec((1, H), lambda i: (0, 0)),
            pl.BlockSpec((H, _PAD), lambda i: (0, 0)),
            pl.BlockSpec((1, _PAD), lambda i: (0, 0)),
        ],
        out_specs=pl.BlockSpec((bm, outw), lambda i: (i, 0)),
        out_shape=jax.ShapeDtypeStruct((B, outw), f32),
        compiler_params=pltpu.CompilerParams(
            dimension_semantics=("parallel",)),
    )(obs, prev_row, W1.astype(bf16), b1.reshape(1, H),
      W2.astype(bf16), b2.reshape(1, H), W3T, tb)
    return out


# speedup vs baseline: 1.6492x; 1.3683x over previous
import jax
import jax.numpy as jnp
from jax.experimental import pallas as pl
from jax.experimental.pallas import tpu as pltpu

_ACTION_S_IDX = 68
_ACTION_E_IDX = 88
_FULL_ACTION_DIM = 28
_BM = 2048


def _onehot_cols(idx_row, nfull, ncols):
    io_p = jax.lax.broadcasted_iota(jnp.int32, (nfull, ncols), 0)
    idx_b = jnp.broadcast_to(idx_row, (nfull, ncols))
    return (idx_b == io_p).astype(jnp.float32)


def _fused_body(obs_ref, prev_ref, w1_ref, b1_ref, w2_ref, b2_ref, w3_ref,
                b3_ref, waoi_ref, woi_ref, offs_ref, defs_ref, keep_ref,
                out_ref):
    f32 = jnp.float32
    bf16 = jnp.bfloat16
    nact = w3_ref.shape[1]
    nfull = _FULL_ACTION_DIM
    outw = out_ref.shape[1]
    dimn = (((0,), (0,)), ((), ()))

    keep24 = jnp.concatenate(
        [keep_ref[...], jnp.full((1, outw - keep_ref.shape[1]), -1, jnp.int32)],
        axis=1)
    OW = _onehot_cols(waoi_ref[...], nfull, nact)
    OD = _onehot_cols(woi_ref[...], nfull, nact)
    OK = _onehot_cols(keep24, nfull, outw)
    A = jax.lax.dot_general(OW, OK, dimn, preferred_element_type=f32)
    AD = jax.lax.dot_general(OD, OK, dimn, preferred_element_type=f32)
    hit = jnp.sum(A, axis=0, keepdims=True)
    cG = (jnp.dot(offs_ref[...], A, preferred_element_type=f32)
          + (1.0 - hit) * jnp.dot(defs_ref[...], AD,
                                  preferred_element_type=f32))
    A01 = A * 0.1
    w3t = jnp.dot(w3_ref[...], A01, preferred_element_type=f32)
    tb = jnp.dot(b3_ref[...], A01, preferred_element_type=f32) + cG

    io_r = jax.lax.broadcasted_iota(jnp.int32, (nact, obs_ref.shape[1]), 0)
    io_c = jax.lax.broadcasted_iota(jnp.int32, (nact, obs_ref.shape[1]), 1)
    SH = (io_c == io_r + _ACTION_S_IDX).astype(f32)
    prev_row = jnp.dot(prev_ref[...], SH, preferred_element_type=f32)

    obs = obs_ref[...]
    col = jax.lax.broadcasted_iota(jnp.int32, obs.shape, 1)
    in_seg = (col >= _ACTION_S_IDX) & (col < _ACTION_E_IDX)
    x = jnp.where(in_seg, prev_row, obs).astype(bf16)
    h = jnp.tanh(jnp.dot(x, w1_ref[...].astype(bf16),
                         preferred_element_type=f32) + b1_ref[...])
    h = jnp.tanh(jnp.dot(h.astype(bf16), w2_ref[...].astype(bf16),
                         preferred_element_type=f32) + b2_ref[...])
    res = jnp.dot(h.astype(bf16), w3t.astype(bf16),
                  preferred_element_type=f32) + tb
    out_ref[...] = res


def kernel(obs, prev_full_action_wk, W1, b1, W2, b2, W3, b3,
           walking_action_out_indices, walking_offsets_indices,
           walking_offsets, walking_defaults, keep_mask):
    B, D = obs.shape
    H = W1.shape[1]
    nact = W3.shape[1]
    nkeep = keep_mask.shape[0]
    outw = nkeep + 4
    row = lambda v: v.reshape(1, -1)

    bm = min(_BM, B)
    full = lambda i: (0, 0)
    out = pl.pallas_call(
        _fused_body,
        grid=(pl.cdiv(B, bm),),
        in_specs=[
            pl.BlockSpec((bm, D), lambda i: (i, 0)),
            pl.BlockSpec((1, nact), full),
            pl.BlockSpec((D, H), full),
            pl.BlockSpec((1, H), full),
            pl.BlockSpec((H, H), full),
            pl.BlockSpec((1, H), full),
            pl.BlockSpec((H, nact), full),
            pl.BlockSpec((1, nact), full),
            pl.BlockSpec((1, nact), full),
            pl.BlockSpec((1, nact), full),
            pl.BlockSpec((1, nact), full),
            pl.BlockSpec((1, nact), full),
            pl.BlockSpec((1, nkeep), full),
        ],
        out_specs=pl.BlockSpec((bm, outw), lambda i: (i, 0)),
        out_shape=jax.ShapeDtypeStruct((B, outw), jnp.float32),
        compiler_params=pltpu.CompilerParams(
            dimension_semantics=("parallel",)),
    )(obs, prev_full_action_wk, W1, row(b1), W2, row(b2), W3, row(b3),
      row(walking_action_out_indices), row(walking_offsets_indices),
      row(walking_offsets), row(walking_defaults), row(keep_mask))
    return out


# speedup vs baseline: 1.6505x; 1.0008x over previous
import jax
import jax.numpy as jnp
from jax.experimental import pallas as pl
from jax.experimental.pallas import tpu as pltpu

_ACTION_S_IDX = 68
_ACTION_E_IDX = 88
_FULL_ACTION_DIM = 28
_BM = 2048


def _onehot_cols(idx_row, nfull, ncols):
    io_p = jax.lax.broadcasted_iota(jnp.int32, (nfull, ncols), 0)
    idx_b = jnp.broadcast_to(idx_row, (nfull, ncols))
    return (idx_b == io_p).astype(jnp.float32)


def _fused_body(obs_ref, prev_ref, w1_ref, b1_ref, w2_ref, b2_ref, w3_ref,
                b3_ref, waoi_ref, woi_ref, offs_ref, defs_ref, keep_ref,
                out_ref):
    f32 = jnp.float32
    bf16 = jnp.bfloat16
    nact = w3_ref.shape[1]
    nfull = _FULL_ACTION_DIM
    outw = out_ref.shape[1]
    dimn = (((0,), (0,)), ((), ()))

    keep24 = jnp.concatenate(
        [keep_ref[...], jnp.full((1, outw - keep_ref.shape[1]), -1, jnp.int32)],
        axis=1)
    OW = _onehot_cols(waoi_ref[...], nfull, nact)
    OD = _onehot_cols(woi_ref[...], nfull, nact)
    OK = _onehot_cols(keep24, nfull, outw)
    A = jax.lax.dot_general(OW, OK, dimn, preferred_element_type=f32)
    AD = jax.lax.dot_general(OD, OK, dimn, preferred_element_type=f32)
    hit = jnp.sum(A, axis=0, keepdims=True)
    cG = (jnp.dot(offs_ref[...], A, preferred_element_type=f32)
          + (1.0 - hit) * jnp.dot(defs_ref[...], AD,
                                  preferred_element_type=f32))
    A01 = A * 0.1
    w3t = jnp.dot(w3_ref[...], A01, preferred_element_type=f32)
    tb = jnp.dot(b3_ref[...], A01, preferred_element_type=f32) + cG

    io_r = jax.lax.broadcasted_iota(jnp.int32, (nact, obs_ref.shape[1]), 0)
    io_c = jax.lax.broadcasted_iota(jnp.int32, (nact, obs_ref.shape[1]), 1)
    SH = (io_c == io_r + _ACTION_S_IDX).astype(f32)
    prev_row = jnp.dot(prev_ref[...], SH, preferred_element_type=f32)

    obs = obs_ref[...]
    col = jax.lax.broadcasted_iota(jnp.int32, obs.shape, 1)
    in_seg = (col >= _ACTION_S_IDX) & (col < _ACTION_E_IDX)
    x = jnp.where(in_seg, prev_row, obs).astype(bf16)
    b1h = b1_ref[...].astype(bf16)
    b2h = b2_ref[...].astype(bf16)
    h = jnp.tanh(jnp.dot(x, w1_ref[...].astype(bf16),
                         preferred_element_type=f32).astype(bf16) + b1h)
    h = jnp.tanh(jnp.dot(h, w2_ref[...].astype(bf16),
                         preferred_element_type=f32).astype(bf16) + b2h)
    res = jnp.dot(h, w3t.astype(bf16),
                  preferred_element_type=f32) + tb
    out_ref[...] = res


def kernel(obs, prev_full_action_wk, W1, b1, W2, b2, W3, b3,
           walking_action_out_indices, walking_offsets_indices,
           walking_offsets, walking_defaults, keep_mask):
    B, D = obs.shape
    H = W1.shape[1]
    nact = W3.shape[1]
    nkeep = keep_mask.shape[0]
    outw = nkeep + 4
    row = lambda v: v.reshape(1, -1)

    bm = min(_BM, B)
    full = lambda i: (0, 0)
    out = pl.pallas_call(
        _fused_body,
        grid=(pl.cdiv(B, bm),),
        in_specs=[
            pl.BlockSpec((bm, D), lambda i: (i, 0)),
            pl.BlockSpec((1, nact), full),
            pl.BlockSpec((D, H), full),
            pl.BlockSpec((1, H), full),
            pl.BlockSpec((H, H), full),
            pl.BlockSpec((1, H), full),
            pl.BlockSpec((H, nact), full),
            pl.BlockSpec((1, nact), full),
            pl.BlockSpec((1, nact), full),
            pl.BlockSpec((1, nact), full),
            pl.BlockSpec((1, nact), full),
            pl.BlockSpec((1, nact), full),
            pl.BlockSpec((1, nkeep), full),
        ],
        out_specs=pl.BlockSpec((bm, outw), lambda i: (i, 0)),
        out_shape=jax.ShapeDtypeStruct((B, outw), jnp.float32),
        compiler_params=pltpu.CompilerParams(
            dimension_semantics=("parallel",)),
    )(obs, prev_full_action_wk, W1, row(b1), W2, row(b2), W3, row(b3),
      row(walking_action_out_indices), row(walking_offsets_indices),
      row(walking_offsets), row(walking_defaults), row(keep_mask))
    return out


# speedup vs baseline: 1.7183x; 1.0411x over previous
import jax
import jax.numpy as jnp
from jax.experimental import pallas as pl
from jax.experimental.pallas import tpu as pltpu

_ACTION_S_IDX = 68
_ACTION_E_IDX = 88
_FULL_ACTION_DIM = 28
_BM = 4096


def _onehot_cols(idx_row, nfull, ncols):
    io_p = jax.lax.broadcasted_iota(jnp.int32, (nfull, ncols), 0)
    idx_b = jnp.broadcast_to(idx_row, (nfull, ncols))
    return (idx_b == io_p).astype(jnp.float32)


def _fused_body(obs_ref, prev_ref, w1_ref, b1_ref, w2_ref, b2_ref, w3_ref,
                b3_ref, waoi_ref, woi_ref, offs_ref, defs_ref, keep_ref,
                out_ref):
    f32 = jnp.float32
    bf16 = jnp.bfloat16
    nact = w3_ref.shape[1]
    nfull = _FULL_ACTION_DIM
    outw = out_ref.shape[1]
    dimn = (((0,), (0,)), ((), ()))

    keep24 = jnp.concatenate(
        [keep_ref[...], jnp.full((1, outw - keep_ref.shape[1]), -1, jnp.int32)],
        axis=1)
    OW = _onehot_cols(waoi_ref[...], nfull, nact)
    OD = _onehot_cols(woi_ref[...], nfull, nact)
    OK = _onehot_cols(keep24, nfull, outw)
    A = jax.lax.dot_general(OW, OK, dimn, preferred_element_type=f32)
    AD = jax.lax.dot_general(OD, OK, dimn, preferred_element_type=f32)
    hit = jnp.sum(A, axis=0, keepdims=True)
    cG = (jnp.dot(offs_ref[...], A, preferred_element_type=f32)
          + (1.0 - hit) * jnp.dot(defs_ref[...], AD,
                                  preferred_element_type=f32))
    A01 = A * 0.1
    w3t = jnp.dot(w3_ref[...], A01, preferred_element_type=f32)
    tb = jnp.dot(b3_ref[...], A01, preferred_element_type=f32) + cG

    io_r = jax.lax.broadcasted_iota(jnp.int32, (nact, obs_ref.shape[1]), 0)
    io_c = jax.lax.broadcasted_iota(jnp.int32, (nact, obs_ref.shape[1]), 1)
    SH = (io_c == io_r + _ACTION_S_IDX).astype(f32)
    prev_row = jnp.dot(prev_ref[...], SH, preferred_element_type=f32)

    obs = obs_ref[...]
    col = jax.lax.broadcasted_iota(jnp.int32, obs.shape, 1)
    in_seg = (col >= _ACTION_S_IDX) & (col < _ACTION_E_IDX)
    x = jnp.where(in_seg, prev_row, obs).astype(bf16)
    b1h = b1_ref[...].astype(bf16)
    b2h = b2_ref[...].astype(bf16)
    h = jnp.tanh(jnp.dot(x, w1_ref[...].astype(bf16),
                         preferred_element_type=f32).astype(bf16) + b1h)
    h = jnp.tanh(jnp.dot(h, w2_ref[...].astype(bf16),
                         preferred_element_type=f32).astype(bf16) + b2h)
    res = jnp.dot(h, w3t.astype(bf16),
                  preferred_element_type=f32) + tb
    out_ref[...] = res


def kernel(obs, prev_full_action_wk, W1, b1, W2, b2, W3, b3,
           walking_action_out_indices, walking_offsets_indices,
           walking_offsets, walking_defaults, keep_mask):
    B, D = obs.shape
    H = W1.shape[1]
    nact = W3.shape[1]
    nkeep = keep_mask.shape[0]
    outw = nkeep + 4
    row = lambda v: v.reshape(1, -1)

    bm = min(_BM, B)
    full = lambda i: (0, 0)
    out = pl.pallas_call(
        _fused_body,
        grid=(pl.cdiv(B, bm),),
        in_specs=[
            pl.BlockSpec((bm, D), lambda i: (i, 0)),
            pl.BlockSpec((1, nact), full),
            pl.BlockSpec((D, H), full),
            pl.BlockSpec((1, H), full),
            pl.BlockSpec((H, H), full),
            pl.BlockSpec((1, H), full),
            pl.BlockSpec((H, nact), full),
            pl.BlockSpec((1, nact), full),
            pl.BlockSpec((1, nact), full),
            pl.BlockSpec((1, nact), full),
            pl.BlockSpec((1, nact), full),
            pl.BlockSpec((1, nact), full),
            pl.BlockSpec((1, nkeep), full),
        ],
        out_specs=pl.BlockSpec((bm, outw), lambda i: (i, 0)),
        out_shape=jax.ShapeDtypeStruct((B, outw), jnp.float32),
        compiler_params=pltpu.CompilerParams(
            dimension_semantics=("parallel",)),
    )(obs, prev_full_action_wk, W1, row(b1), W2, row(b2), W3, row(b3),
      row(walking_action_out_indices), row(walking_offsets_indices),
      row(walking_offsets), row(walking_defaults), row(keep_mask))
    return out
